# speedup vs baseline: 10.9138x; 10.9138x over previous
import functools

import jax
import jax.numpy as jnp
from jax import lax
from jax.experimental import pallas as pl
from jax.experimental.pallas import tpu as pltpu
from jax.experimental.pallas import tpu_sc as plsc

N = 10000
E = 320000
D = 128

NPAD = 10240
SENT = 10000
NTILES = 32
CHUNK = 128
EPT = 10112
NCHUNK = EPT // CHUNK
E_PAD = EPT * NTILES
ROWS_PER_TILE = NPAD // 16
R = 1024

_MESH = plsc.VectorSubcoreMesh(core_axis_name="c", subcore_axis_name="s")


def _zero_rows(rows):
    @pl.loop(0, CHUNK)
    def _r(r):
        @pl.loop(0, D // 16)
        def _c(j):
            rows[r, pl.ds(j * 16, 16)] = jnp.zeros((16,), jnp.float32)


@functools.partial(
    pl.kernel,
    out_type=jax.ShapeDtypeStruct((2 * NPAD,), jnp.float32),
    mesh=_MESH,
    scratch_types=[
        pltpu.VMEM((CHUNK,), jnp.int32),
        pltpu.VMEM((CHUNK,), jnp.float32),
        pltpu.VMEM((ROWS_PER_TILE,), jnp.float32),
        pltpu.VMEM_SHARED((NPAD,), jnp.float32),
    ],
)
def _sc_degree(dst_hbm, out_hbm, dstb, ones, zb, deg):
    c = lax.axis_index("c")
    s = lax.axis_index("s")
    wid = c * 16 + s

    @pl.loop(0, CHUNK // 16)
    def _o(i):
        ones[pl.ds(i * 16, 16)] = jnp.full((16,), 1.0, jnp.float32)

    @pl.loop(0, ROWS_PER_TILE // 16)
    def _z(i):
        zb[pl.ds(i * 16, 16)] = jnp.zeros((16,), jnp.float32)

    pltpu.sync_copy(zb, deg.at[pl.ds(s * ROWS_PER_TILE, ROWS_PER_TILE)])
    plsc.subcore_barrier()

    base = wid * EPT

    @pl.loop(0, NCHUNK)
    def _k(k):
        pltpu.sync_copy(dst_hbm.at[pl.ds(base + k * CHUNK, CHUNK)], dstb)
        pltpu.sync_copy(ones, deg.at[dstb], add=True)

    plsc.subcore_barrier()
    pltpu.sync_copy(
        deg.at[pl.ds(s * ROWS_PER_TILE, ROWS_PER_TILE)],
        out_hbm.at[pl.ds(c * NPAD + s * ROWS_PER_TILE, ROWS_PER_TILE)],
    )


@functools.partial(
    pl.kernel,
    out_type=jax.ShapeDtypeStruct((2 * NPAD, D), jnp.float32),
    mesh=_MESH,
    scratch_types=[
        pltpu.VMEM((CHUNK,), jnp.int32),
        pltpu.VMEM((CHUNK,), jnp.int32),
        pltpu.VMEM((CHUNK, D), jnp.float32),
        pltpu.VMEM_SHARED((NPAD, D), jnp.float32),
        pltpu.SemaphoreType.DMA,
    ],
)
def _sc_segsum(hp_hbm, src_hbm, dst_hbm, out_hbm, srcb, dstb, rows, accum, sem):
    c = lax.axis_index("c")
    s = lax.axis_index("s")
    wid = c * 16 + s

    _zero_rows(rows)

    @pl.loop(0, ROWS_PER_TILE // CHUNK)
    def _z(i):
        pltpu.sync_copy(rows, accum.at[pl.ds(s * ROWS_PER_TILE + i * CHUNK, CHUNK)])

    plsc.subcore_barrier()

    base = wid * EPT

    @pl.loop(0, NCHUNK)
    def _k(k):
        pltpu.sync_copy(src_hbm.at[pl.ds(base + k * CHUNK, CHUNK)], srcb)
        pltpu.sync_copy(dst_hbm.at[pl.ds(base + k * CHUNK, CHUNK)], dstb)
        pltpu.async_copy(hp_hbm.at[srcb], rows, sem).wait()
        pltpu.sync_copy(rows, accum.at[dstb], add=True)

    plsc.subcore_barrier()
    pltpu.sync_copy(
        accum.at[pl.ds(s * ROWS_PER_TILE, ROWS_PER_TILE)],
        out_hbm.at[pl.ds(c * NPAD + s * ROWS_PER_TILE, ROWS_PER_TILE)],
    )


def _tc1_body(x_ref, w_ref, p0_ref, p1_ref, hp_ref, dinv_ref):
    dinv = lax.rsqrt(p0_ref[...] + p1_ref[...] + 1.0)
    h = jnp.dot(x_ref[...], w_ref[...], preferred_element_type=jnp.float32)
    hp_ref[...] = h * dinv
    dinv_ref[...] = dinv


def _tc2_body(s_ref, hp_ref, dinv_ref, b_ref, w_ref, h2p_ref):
    dinv = dinv_ref[...]
    out1 = dinv * (s_ref[0] + s_ref[1] + hp_ref[...]) + b_ref[...]
    h2 = jnp.dot(out1, w_ref[...], preferred_element_type=jnp.float32)
    h2p_ref[...] = h2 * dinv


def _tc3_body(s_ref, hp_ref, dinv_ref, b_ref, out_ref):
    out_ref[...] = (
        dinv_ref[...] * (s_ref[0] + s_ref[1] + hp_ref[...]) + b_ref[...]
    )


_GRID = (NPAD // R,)
_ROWS = pl.BlockSpec((R, D), lambda i: (i, 0))
_COL = pl.BlockSpec((R, 1), lambda i: (i, 0))
_WMAT = pl.BlockSpec((D, D), lambda i: (0, 0))
_BVEC = pl.BlockSpec((1, D), lambda i: (0, 0))
_PART = pl.BlockSpec((2, R, D), lambda i: (0, i, 0))

_tc1 = pl.pallas_call(
    _tc1_body,
    grid=_GRID,
    in_specs=[_ROWS, _WMAT, _COL, _COL],
    out_specs=[_ROWS, _COL],
    out_shape=[
        jax.ShapeDtypeStruct((NPAD, D), jnp.float32),
        jax.ShapeDtypeStruct((NPAD, 1), jnp.float32),
    ],
)

_tc2 = pl.pallas_call(
    _tc2_body,
    grid=_GRID,
    in_specs=[_PART, _ROWS, _COL, _BVEC, _WMAT],
    out_specs=_ROWS,
    out_shape=jax.ShapeDtypeStruct((NPAD, D), jnp.float32),
)

_tc3 = pl.pallas_call(
    _tc3_body,
    grid=_GRID,
    in_specs=[_PART, _ROWS, _COL, _BVEC],
    out_specs=_ROWS,
    out_shape=jax.ShapeDtypeStruct((NPAD, D), jnp.float32),
)


def kernel(x, edge_index, W1, b1, W2, b2):
    src = edge_index[0].astype(jnp.int32)
    dst = edge_index[1].astype(jnp.int32)
    pad = jnp.full((E_PAD - E,), SENT, jnp.int32)
    src_p = jnp.concatenate([src, pad])
    dst_p = jnp.concatenate([dst, pad])
    x_pad = jnp.pad(x, ((0, NPAD - N), (0, 0)))
    b1r = b1.reshape(1, D)
    b2r = b2.reshape(1, D)

    degp = _sc_degree(dst_p)
    p0 = degp[:NPAD].reshape(NPAD, 1)
    p1 = degp[NPAD:].reshape(NPAD, 1)

    hp1, dinv = _tc1(x_pad, W1, p0, p1)
    s1 = _sc_segsum(hp1, src_p, dst_p).reshape(2, NPAD, D)
    h2p = _tc2(s1, hp1, dinv, b1r, W2)
    s2 = _sc_segsum(h2p, src_p, dst_p).reshape(2, NPAD, D)
    out2 = _tc3(s2, h2p, dinv, b2r)
    return out2[:N]
